# baseline (speedup 1.0000x reference)
import functools

import jax
import jax.numpy as jnp
from jax import lax
from jax.experimental import pallas as pl
from jax.experimental.pallas import tpu as pltpu
from jax.experimental.pallas import tpu_sc as plsc

MAXZ = 54
NODE = 128
NW = 32
CHUNK = 128
CHUNKS_PER_W = 25
PER_W = CHUNK * CHUNKS_PER_W
N_OUT = 100000

NSLOT = 4
GRP = CHUNK // 16
CN = CHUNK * NODE

_mesh = plsc.VectorSubcoreMesh(core_axis_name="c", subcore_axis_name="s")


@functools.partial(
    pl.kernel,
    mesh=_mesh,
    out_type=jax.ShapeDtypeStruct((N_OUT * NODE,), jnp.float32),
    scratch_types=[
        pltpu.VMEM((MAXZ * NODE,), jnp.float32),
        pltpu.VMEM((PER_W,), jnp.int32),
        pltpu.VMEM((NSLOT * CN,), jnp.float32),
        pltpu.SemaphoreType.DMA((NSLOT,)),
        pltpu.SemaphoreType.DMA((2,)),
    ],
    compiler_params=pltpu.CompilerParams(needs_layout_passes=False),
)
def _embed_lookup(table_hbm, z_hbm, out_hbm, table_v, idx_v, bufs, ssem, lsem):
    wid = lax.axis_index("s") * 2 + lax.axis_index("c")
    base = jnp.minimum(wid * PER_W, N_OUT - PER_W)
    tcp = pltpu.async_copy(table_hbm, table_v, lsem.at[0])
    icp = pltpu.async_copy(z_hbm.at[pl.ds(base, PER_W)], idx_v, lsem.at[1])
    tcp.wait()
    icp.wait()

    lanes = lax.iota(jnp.int32, 16)
    row_flat = [(lanes + 16 * g) * NODE for g in range(GRP)]

    def chunk_copy(i, b):
        off = (base + i * CHUNK) * NODE
        return pltpu.make_async_copy(
            bufs.at[pl.ds(b * CN, CN)], out_hbm.at[pl.ds(off, CN)], ssem.at[b]
        )

    def body(i, carry):
        b = lax.rem(i, NSLOT)

        @pl.when(i >= NSLOT)
        def _drain():
            chunk_copy(i - NSLOT, b).wait()

        buf = bufs.at[pl.ds(b * CN, CN)]
        zb = [idx_v[pl.ds(i * CHUNK + 16 * g, 16)] * NODE for g in range(GRP)]

        @plsc.parallel_loop(0, NODE, unroll=4, carry=lax.iota(jnp.int32, 16))
        def _cols(c, cvec):
            for g in range(GRP):
                vals = plsc.load_gather(table_v, [zb[g] + cvec])
                plsc.store_scatter(buf, [row_flat[g] + cvec], vals)
            return (cvec + 1) & (NODE - 1)

        chunk_copy(i, b).start()
        return carry

    lax.fori_loop(0, CHUNKS_PER_W, body, 0)

    def drain(i, carry):
        chunk_copy(i, lax.rem(i, NSLOT)).wait()
        return carry

    lax.fori_loop(CHUNKS_PER_W - NSLOT, CHUNKS_PER_W, drain, 0)


def kernel(Z, table):
    out = _embed_lookup(table.reshape(-1), Z.astype(jnp.int32))
    return out.reshape(N_OUT, NODE)


# speedup vs baseline: 1.0374x; 1.0374x over previous
import functools

import jax
import jax.numpy as jnp
from jax import lax
from jax.experimental import pallas as pl
from jax.experimental.pallas import tpu as pltpu
from jax.experimental.pallas import tpu_sc as plsc

MAXZ = 54
NODE = 128
NW = 32
CHUNK = 128
CHUNKS_PER_W = 25
PER_W = CHUNK * CHUNKS_PER_W
N_OUT = 100000

NSLOT = 4
GRP = CHUNK // 16
CN = CHUNK * NODE

_mesh = plsc.VectorSubcoreMesh(core_axis_name="c", subcore_axis_name="s")


@functools.partial(
    pl.kernel,
    mesh=_mesh,
    out_type=jax.ShapeDtypeStruct((N_OUT * NODE,), jnp.float32),
    scratch_types=[
        pltpu.VMEM((MAXZ * NODE,), jnp.float32),
        pltpu.VMEM((PER_W,), jnp.int32),
        pltpu.VMEM((NSLOT * CN,), jnp.float32),
        pltpu.SemaphoreType.DMA((NSLOT,)),
        pltpu.SemaphoreType.DMA((2,)),
    ],
    compiler_params=pltpu.CompilerParams(needs_layout_passes=False),
)
def _embed_lookup(table_hbm, z_hbm, out_hbm, table_v, idx_v, bufs, ssem, lsem):
    wid = lax.axis_index("s") * 2 + lax.axis_index("c")
    base = jnp.minimum(wid * PER_W, N_OUT - PER_W)
    tcp = pltpu.async_copy(table_hbm, table_v, lsem.at[0])
    icp = pltpu.async_copy(z_hbm.at[pl.ds(base, PER_W)], idx_v, lsem.at[1])
    tcp.wait()
    icp.wait()

    lanes = lax.iota(jnp.int32, 16)
    row_flat = [(lanes + 16 * g) * NODE for g in range(GRP)]

    def chunk_copy(i, b):
        off = (base + i * CHUNK) * NODE
        return pltpu.make_async_copy(
            bufs.at[pl.ds(b * CN, CN)], out_hbm.at[pl.ds(off, CN)], ssem.at[b]
        )

    def body(i, carry):
        b = lax.rem(i, NSLOT)

        @pl.when(i >= NSLOT)
        def _drain():
            chunk_copy(i - NSLOT, b).wait()

        buf = bufs.at[pl.ds(b * CN, CN)]
        zb = [idx_v[pl.ds(i * CHUNK + 16 * g, 16)] * NODE for g in range(GRP)]

        @plsc.parallel_loop(0, NODE, unroll=2, carry=lax.iota(jnp.int32, 16))
        def _cols(c, cvec):
            for g in range(GRP):
                vals = plsc.load_gather(table_v, [zb[g] + cvec])
                plsc.store_scatter(buf, [row_flat[g] + cvec], vals)
            return (cvec + 1) & (NODE - 1)

        chunk_copy(i, b).start()
        return carry

    lax.fori_loop(0, CHUNKS_PER_W, body, 0)

    def drain(i, carry):
        chunk_copy(i, lax.rem(i, NSLOT)).wait()
        return carry

    lax.fori_loop(CHUNKS_PER_W - NSLOT, CHUNKS_PER_W, drain, 0)


def kernel(Z, table):
    out = _embed_lookup(table.reshape(-1), Z.astype(jnp.int32))
    return out.reshape(N_OUT, NODE)
